# Initial kernel scaffold; baseline (speedup 1.0000x reference)
import jax
import jax.numpy as jnp
from jax.experimental import pallas as pl


def kernel(x, edge_index, W1, b1, W2, b2):
    raise NotImplementedError("write your pallas kernel here")


# speedup vs baseline: 13.5287x; 13.5287x over previous
import jax
import jax.numpy as jnp
from jax import lax
from jax.experimental import pallas as pl
from jax.experimental.pallas import tpu as pltpu
from jax.experimental.pallas import tpu_sc as plsc

N = 10000
D = 128
NC, NS = 2, 16
NW = NC * NS
NP = 10240
RPT = NP // NS
K = 128
GB = NP // D


def _count_body(dst_hbm, ones_hbm, zn_hbm, cnt_hbm, cnt_sh, dst_v, ones_v):
    c = lax.axis_index("c")
    s = lax.axis_index("s")
    ep = dst_hbm.shape[0]
    ept = ep // NW
    pltpu.sync_copy(ones_hbm, ones_v)
    pltpu.sync_copy(zn_hbm, cnt_sh.at[pl.ds(s * RPT, RPT)])
    plsc.subcore_barrier()
    base = c * (ep // NC) + s * ept

    def body(i, carry):
        pltpu.sync_copy(dst_hbm.at[pl.ds(base + i * K, K)], dst_v)
        pltpu.sync_copy(ones_v, cnt_sh.at[dst_v], add=True)
        return carry

    lax.fori_loop(0, ept // K, body, 0)
    plsc.subcore_barrier()
    pltpu.sync_copy(cnt_sh.at[pl.ds(s * RPT, RPT)],
                    cnt_hbm.at[pl.ds(c * NP + s * RPT, RPT)])


def _edge_body(g_hbm, src_hbm, dst_hbm, znd_hbm, acc_hbm,
               acc_sh, src_v, dst_v, rows_v, sem):
    c = lax.axis_index("c")
    s = lax.axis_index("s")
    ep = src_hbm.shape[0]
    ept = ep // NW
    pltpu.sync_copy(znd_hbm, acc_sh.at[pl.ds(s * RPT, RPT)])
    plsc.subcore_barrier()
    base = c * (ep // NC) + s * ept

    def body(i, carry):
        off = base + i * K
        pltpu.sync_copy(src_hbm.at[pl.ds(off, K)], src_v)
        pltpu.sync_copy(dst_hbm.at[pl.ds(off, K)], dst_v)
        pltpu.async_copy(g_hbm.at[src_v], rows_v, sem).wait()
        pltpu.sync_copy(rows_v, acc_sh.at[dst_v], add=True)
        return carry

    lax.fori_loop(0, ept // K, body, 0)
    plsc.subcore_barrier()
    pltpu.sync_copy(acc_sh.at[pl.ds(s * RPT, RPT)],
                    acc_hbm.at[pl.ds(c * NP + s * RPT, RPT)])


def _dinv_mat(cnt_ref, i):
    cz = cnt_ref[...]
    deg = (cz[0] + cz[1]).reshape(1, D) + 1.0
    lanes = lax.broadcasted_iota(jnp.int32, (1, D), 1)
    valid = (i * D + lanes) < N
    dv = jnp.where(valid, lax.rsqrt(deg), 0.0)
    ones = jnp.ones((1, D), jnp.float32)
    return lax.dot_general(dv, ones, (((0,), (0,)), ((), ())),
                           preferred_element_type=jnp.float32)


def _mm1_body(cnt_ref, x_ref, w_ref, g_ref):
    i = pl.program_id(0)
    dm = _dinv_mat(cnt_ref, i)
    xw = jnp.dot(x_ref[...], w_ref[...], preferred_element_type=jnp.float32)
    g_ref[...] = dm * xw


def _mm2_body(cnt_ref, acc_ref, g1_ref, w_ref, b_ref, g2_ref):
    i = pl.program_id(0)
    dm = _dinv_mat(cnt_ref, i)
    a = acc_ref[...]
    pre = dm * (a[0] + a[1] + g1_ref[...]) + b_ref[...]
    h = jnp.maximum(pre, 0.0)
    hw = jnp.dot(h, w_ref[...], preferred_element_type=jnp.float32)
    g2_ref[...] = dm * hw


def _fin_body(cnt_ref, acc_ref, g2_ref, b_ref, out_ref):
    i = pl.program_id(0)
    dm = _dinv_mat(cnt_ref, i)
    a = acc_ref[...]
    out_ref[...] = dm * (a[0] + a[1] + g2_ref[...]) + b_ref[...]


def kernel(x, edge_index, W1, b1, W2, b2):
    ei = edge_index.astype(jnp.int32)
    src, dst = ei[0], ei[1]
    e = src.shape[0]
    chunk = NW * K
    ep = ((e + chunk - 1) // chunk) * chunk
    pad = ep - e
    padidx = N + (jnp.arange(pad, dtype=jnp.int32) % (NP - N))
    srcp = jnp.concatenate([src, padidx])
    dstp = jnp.concatenate([dst, padidx])
    xp = jnp.pad(x, ((0, NP - N), (0, 0)))
    ones_k = jnp.ones((K,), jnp.float32)
    zn = jnp.zeros((RPT,), jnp.float32)
    znd = jnp.zeros((RPT, D), jnp.float32)

    mesh = plsc.VectorSubcoreMesh(core_axis_name="c", subcore_axis_name="s")

    cnt = pl.kernel(
        _count_body,
        out_type=jax.ShapeDtypeStruct((NC * NP,), jnp.float32),
        mesh=mesh,
        scratch_types=[
            pltpu.VMEM_SHARED((NP,), jnp.float32),
            pltpu.VMEM((K,), jnp.int32),
            pltpu.VMEM((K,), jnp.float32),
        ],
    )(dstp, ones_k, zn)
    cntr = cnt.reshape(NC, GB, 1, D)

    edge_call = pl.kernel(
        _edge_body,
        out_type=jax.ShapeDtypeStruct((NC * NP, D), jnp.float32),
        mesh=mesh,
        scratch_types=[
            pltpu.VMEM_SHARED((NP, D), jnp.float32),
            pltpu.VMEM((K,), jnp.int32),
            pltpu.VMEM((K,), jnp.int32),
            pltpu.VMEM((K, D), jnp.float32),
            pltpu.SemaphoreType.DMA,
        ],
    )

    g1 = pl.pallas_call(
        _mm1_body,
        grid=(GB,),
        in_specs=[
            pl.BlockSpec((NC, 1, 1, D), lambda i: (0, i, 0, 0)),
            pl.BlockSpec((D, D), lambda i: (i, 0)),
            pl.BlockSpec((D, D), lambda i: (0, 0)),
        ],
        out_specs=pl.BlockSpec((D, D), lambda i: (i, 0)),
        out_shape=jax.ShapeDtypeStruct((NP, D), jnp.float32),
    )(cntr, xp, W1)

    acc1 = edge_call(g1, srcp, dstp, znd).reshape(NC, NP, D)

    g2 = pl.pallas_call(
        _mm2_body,
        grid=(GB,),
        in_specs=[
            pl.BlockSpec((NC, 1, 1, D), lambda i: (0, i, 0, 0)),
            pl.BlockSpec((NC, D, D), lambda i: (0, i, 0)),
            pl.BlockSpec((D, D), lambda i: (i, 0)),
            pl.BlockSpec((D, D), lambda i: (0, 0)),
            pl.BlockSpec((1, D), lambda i: (0, 0)),
        ],
        out_specs=pl.BlockSpec((D, D), lambda i: (i, 0)),
        out_shape=jax.ShapeDtypeStruct((NP, D), jnp.float32),
    )(cntr, acc1, g1, W2, b1.reshape(1, D))

    acc2 = edge_call(g2, srcp, dstp, znd).reshape(NC, NP, D)

    out = pl.pallas_call(
        _fin_body,
        grid=(GB,),
        in_specs=[
            pl.BlockSpec((NC, 1, 1, D), lambda i: (0, i, 0, 0)),
            pl.BlockSpec((NC, D, D), lambda i: (0, i, 0)),
            pl.BlockSpec((D, D), lambda i: (i, 0)),
            pl.BlockSpec((1, D), lambda i: (0, 0)),
        ],
        out_specs=pl.BlockSpec((D, D), lambda i: (i, 0)),
        out_shape=jax.ShapeDtypeStruct((NP, D), jnp.float32),
    )(cntr, acc2, g2, b2.reshape(1, D))

    return out[:N]


# speedup vs baseline: 21.5946x; 1.5962x over previous
import jax
import jax.numpy as jnp
from jax import lax
from jax.experimental import pallas as pl
from jax.experimental.pallas import tpu as pltpu
from jax.experimental.pallas import tpu_sc as plsc

N = 10000
D = 128
NC, NS = 2, 16
NW = NC * NS
NP = 10240
RPT = NP // NS
K = 128
CH = 80
GB = NP // D


def _count_body(sd_hbm, ones_hbm, zn_hbm, cnt_hbm, cnt_sh, sdb, ones_v, sem):
    c = lax.axis_index("c")
    s = lax.axis_index("s")
    w = c * NS + s
    pltpu.sync_copy(sd_hbm.at[w], sdb)
    pltpu.sync_copy(ones_hbm, ones_v)
    pltpu.sync_copy(zn_hbm, cnt_sh.at[pl.ds(s * RPT, RPT)])
    plsc.subcore_barrier()
    gf = 16

    def body(j, carry):
        for t in range(gf):
            pltpu.async_copy(ones_v, cnt_sh.at[sdb.at[j * gf + t, 1]], sem,
                             add=True)
        for t in range(gf):
            pltpu.make_async_copy(ones_v, cnt_sh.at[sdb.at[0, 1]], sem).wait()
        return carry

    lax.fori_loop(0, CH // gf, body, 0)
    plsc.subcore_barrier()
    pltpu.sync_copy(cnt_sh.at[pl.ds(s * RPT, RPT)],
                    cnt_hbm.at[pl.ds(c * NP + s * RPT, RPT)])


def _edge_body(g_hbm, sd_hbm, znd_hbm, acc_hbm,
               acc_sh, idxb, rows, sg0, sg1, ss0, ss1, sx0, sx1, sx2, sx3):
    c = lax.axis_index("c")
    s = lax.axis_index("s")
    w = c * NS + s
    sg = [sg0, sg1]
    ss = [ss0, ss1]
    sx = [sx0, sx1, sx2, sx3]

    def xfire(i, q):
        pltpu.async_copy(sd_hbm.at[w, i], idxb.at[q], sx[q])

    def xwait(q):
        pltpu.make_async_copy(sd_hbm.at[0, 0], idxb.at[q], sx[q]).wait()

    def gfire(b, q):
        pltpu.async_copy(g_hbm.at[idxb.at[q, 0]], rows.at[b], sg[b])

    def gwait(b):
        pltpu.make_async_copy(g_hbm.at[idxb.at[0, 0]], rows.at[b],
                              sg[b]).wait()

    def sfire(b, q):
        pltpu.async_copy(rows.at[b], acc_sh.at[idxb.at[q, 1]], ss[b],
                         add=True)

    def swait(b):
        pltpu.make_async_copy(rows.at[b], acc_sh.at[idxb.at[0, 1]],
                              ss[b]).wait()

    pltpu.sync_copy(znd_hbm, acc_sh.at[pl.ds(s * RPT, RPT)])
    xfire(0, 0)
    plsc.subcore_barrier()

    def body(j, carry):
        for u in range(4):
            i = j * 4 + u
            b = u % 2
            q = u
            qm = (u + 3) % 4

            if u >= 2:
                swait(b)
            else:
                @pl.when(j >= 1)
                def _():
                    swait(b)

            if u < 3:
                xfire(i + 1, u + 1)
            else:
                @pl.when(j <= CH // 4 - 2)
                def _():
                    xfire(i + 1, 0)

            if u >= 1:
                gwait(1 - b)
                sfire(1 - b, qm)
            else:
                @pl.when(j >= 1)
                def _():
                    gwait(1 - b)
                    sfire(1 - b, qm)

            xwait(q)
            gfire(b, q)
        return carry

    lax.fori_loop(0, CH // 4, body, 0)
    gwait(1)
    sfire(1, 3)
    swait(0)
    swait(1)
    plsc.subcore_barrier()
    pltpu.sync_copy(acc_sh.at[pl.ds(s * RPT, RPT)],
                    acc_hbm.at[pl.ds(c * NP + s * RPT, RPT)])


def _dinv_mat(cnt_ref, i):
    cz = cnt_ref[...]
    deg = (cz[0] + cz[1]).reshape(1, D) + 1.0
    lanes = lax.broadcasted_iota(jnp.int32, (1, D), 1)
    valid = (i * D + lanes) < N
    dv = jnp.where(valid, lax.rsqrt(deg), 0.0)
    ones = jnp.ones((1, D), jnp.float32)
    return lax.dot_general(dv, ones, (((0,), (0,)), ((), ())),
                           preferred_element_type=jnp.float32)


def _mm1_body(cnt_ref, x_ref, w_ref, g_ref):
    i = pl.program_id(0)
    dm = _dinv_mat(cnt_ref, i)
    xw = jnp.dot(x_ref[...], w_ref[...], preferred_element_type=jnp.float32)
    g_ref[...] = dm * xw


def _mm2_body(cnt_ref, acc_ref, g1_ref, w_ref, b_ref, g2_ref):
    i = pl.program_id(0)
    dm = _dinv_mat(cnt_ref, i)
    a = acc_ref[...]
    pre = dm * (a[0] + a[1] + g1_ref[...]) + b_ref[...]
    h = jnp.maximum(pre, 0.0)
    hw = jnp.dot(h, w_ref[...], preferred_element_type=jnp.float32)
    g2_ref[...] = dm * hw


def _fin_body(cnt_ref, acc_ref, g2_ref, b_ref, out_ref):
    i = pl.program_id(0)
    dm = _dinv_mat(cnt_ref, i)
    a = acc_ref[...]
    out_ref[...] = dm * (a[0] + a[1] + g2_ref[...]) + b_ref[...]


def kernel(x, edge_index, W1, b1, W2, b2):
    ei = edge_index.astype(jnp.int32)
    src, dst = ei[0], ei[1]
    e = src.shape[0]
    ep = NW * CH * K
    pad = ep - e
    padidx = N + (jnp.arange(pad, dtype=jnp.int32) % (NP - N))
    srcp = jnp.concatenate([src, padidx]).reshape(NW, CH, 1, K)
    dstp = jnp.concatenate([dst, padidx]).reshape(NW, CH, 1, K)
    sd = jnp.concatenate([srcp, dstp], axis=2)
    xp = jnp.pad(x, ((0, NP - N), (0, 0)))
    ones_k = jnp.ones((K,), jnp.float32)
    zn = jnp.zeros((RPT,), jnp.float32)
    znd = jnp.zeros((RPT, D), jnp.float32)

    mesh = plsc.VectorSubcoreMesh(core_axis_name="c", subcore_axis_name="s")

    cnt = pl.kernel(
        _count_body,
        out_type=jax.ShapeDtypeStruct((NC * NP,), jnp.float32),
        mesh=mesh,
        scratch_types=[
            pltpu.VMEM_SHARED((NP,), jnp.float32),
            pltpu.VMEM((CH, 2, K), jnp.int32),
            pltpu.VMEM((K,), jnp.float32),
            pltpu.SemaphoreType.DMA,
        ],
    )(sd, ones_k, zn)
    cntr = cnt.reshape(NC, GB, 1, D)

    edge_call = pl.kernel(
        _edge_body,
        out_type=jax.ShapeDtypeStruct((NC * NP, D), jnp.float32),
        mesh=mesh,
        scratch_types=[
            pltpu.VMEM_SHARED((NP, D), jnp.float32),
            pltpu.VMEM((4, 2, K), jnp.int32),
            pltpu.VMEM((2, K, D), jnp.float32),
        ] + [pltpu.SemaphoreType.DMA] * 8,
    )

    g1 = pl.pallas_call(
        _mm1_body,
        grid=(GB,),
        in_specs=[
            pl.BlockSpec((NC, 1, 1, D), lambda i: (0, i, 0, 0)),
            pl.BlockSpec((D, D), lambda i: (i, 0)),
            pl.BlockSpec((D, D), lambda i: (0, 0)),
        ],
        out_specs=pl.BlockSpec((D, D), lambda i: (i, 0)),
        out_shape=jax.ShapeDtypeStruct((NP, D), jnp.float32),
    )(cntr, xp, W1)

    acc1 = edge_call(g1, sd, znd).reshape(NC, NP, D)

    g2 = pl.pallas_call(
        _mm2_body,
        grid=(GB,),
        in_specs=[
            pl.BlockSpec((NC, 1, 1, D), lambda i: (0, i, 0, 0)),
            pl.BlockSpec((NC, D, D), lambda i: (0, i, 0)),
            pl.BlockSpec((D, D), lambda i: (i, 0)),
            pl.BlockSpec((D, D), lambda i: (0, 0)),
            pl.BlockSpec((1, D), lambda i: (0, 0)),
        ],
        out_specs=pl.BlockSpec((D, D), lambda i: (i, 0)),
        out_shape=jax.ShapeDtypeStruct((NP, D), jnp.float32),
    )(cntr, acc1, g1, W2, b1.reshape(1, D))

    acc2 = edge_call(g2, sd, znd).reshape(NC, NP, D)

    out = pl.pallas_call(
        _fin_body,
        grid=(GB,),
        in_specs=[
            pl.BlockSpec((NC, 1, 1, D), lambda i: (0, i, 0, 0)),
            pl.BlockSpec((NC, D, D), lambda i: (0, i, 0)),
            pl.BlockSpec((D, D), lambda i: (i, 0)),
            pl.BlockSpec((1, D), lambda i: (0, 0)),
        ],
        out_specs=pl.BlockSpec((D, D), lambda i: (i, 0)),
        out_shape=jax.ShapeDtypeStruct((NP, D), jnp.float32),
    )(cntr, acc2, g2, b2.reshape(1, D))

    return out[:N]


# speedup vs baseline: 34.4426x; 1.5950x over previous
import jax
import jax.numpy as jnp
from jax import lax
from jax.experimental import pallas as pl
from jax.experimental.pallas import tpu as pltpu
from jax.experimental.pallas import tpu_sc as plsc

N = 10000
D = 128
NC, NS = 2, 16
NW = NC * NS
NP = 10240
RPT = NP // NS
K = 128
CH = 80
KC = 80
CHC = 125
RB = 2048
NRB = NP // RB


def _count_body(dst2_hbm, ones_hbm, zn_hbm, cnt_hbm, cnt_sh, dstb, ones_v,
                sem):
    c = lax.axis_index("c")
    s = lax.axis_index("s")
    w = c * NS + s
    pltpu.sync_copy(dst2_hbm.at[w], dstb)
    pltpu.sync_copy(ones_hbm, ones_v)
    pltpu.sync_copy(zn_hbm, cnt_sh.at[pl.ds(s * RPT, RPT)])
    plsc.subcore_barrier()
    gf = 25

    def body(j, carry):
        for t in range(gf):
            pltpu.async_copy(ones_v, cnt_sh.at[dstb.at[j * gf + t]], sem,
                             add=True)
        for t in range(gf):
            pltpu.make_async_copy(ones_v, cnt_sh.at[dstb.at[0]], sem).wait()
        return carry

    lax.fori_loop(0, CHC // gf, body, 0)
    plsc.subcore_barrier()
    pltpu.sync_copy(cnt_sh.at[pl.ds(s * RPT, RPT)],
                    cnt_hbm.at[pl.ds(c * NP + s * RPT, RPT)])


def _edge_body(g_hbm, src_hbm, dst_hbm, znd_hbm, acc_hbm,
               acc_sh, idxb, rows, sg0, sg1, ss0, ss1, sx0, sx1, sx2, sx3):
    c = lax.axis_index("c")
    s = lax.axis_index("s")
    w = c * NS + s
    sg = [sg0, sg1]
    ss = [ss0, ss1]
    sx = [sx0, sx1, sx2, sx3]

    def xfire(i, q):
        pltpu.async_copy(src_hbm.at[w, i], idxb.at[q, 0], sx[q])
        pltpu.async_copy(dst_hbm.at[w, i], idxb.at[q, 1], sx[q])

    def xwait(q):
        pltpu.make_async_copy(src_hbm.at[0, 0], idxb.at[q, 0], sx[q]).wait()
        pltpu.make_async_copy(dst_hbm.at[0, 0], idxb.at[q, 1], sx[q]).wait()

    def gfire(b, q):
        pltpu.async_copy(g_hbm.at[idxb.at[q, 0]], rows.at[b], sg[b])

    def gwait(b):
        pltpu.make_async_copy(g_hbm.at[idxb.at[0, 0]], rows.at[b],
                              sg[b]).wait()

    def sfire(b, q):
        pltpu.async_copy(rows.at[b], acc_sh.at[idxb.at[q, 1]], ss[b],
                         add=True)

    def swait(b):
        pltpu.make_async_copy(rows.at[b], acc_sh.at[idxb.at[0, 1]],
                              ss[b]).wait()

    pltpu.sync_copy(znd_hbm, acc_sh.at[pl.ds(s * RPT, RPT)])
    xfire(0, 0)
    plsc.subcore_barrier()

    def body(j, carry):
        for u in range(4):
            i = j * 4 + u
            b = u % 2
            q = u
            qm = (u + 3) % 4

            if u >= 2:
                swait(b)
            else:
                @pl.when(j >= 1)
                def _():
                    swait(b)

            xwait(q)
            gfire(b, q)

            if u < 3:
                xfire(i + 1, u + 1)
            else:
                @pl.when(j <= CH // 4 - 2)
                def _():
                    xfire(i + 1, 0)

            if u >= 1:
                gwait(1 - b)
                sfire(1 - b, qm)
            else:
                @pl.when(j >= 1)
                def _():
                    gwait(1 - b)
                    sfire(1 - b, qm)
        return carry

    lax.fori_loop(0, CH // 4, body, 0)
    gwait(1)
    sfire(1, 3)
    swait(0)
    swait(1)
    plsc.subcore_barrier()
    pltpu.sync_copy(acc_sh.at[pl.ds(s * RPT, RPT)],
                    acc_hbm.at[pl.ds(c * NP + s * RPT, RPT)])


def _dinv_mat(cnt_ref, i):
    cz = cnt_ref[...]
    deg = (cz[0] + cz[1]).reshape(1, RB) + 1.0
    lanes = lax.broadcasted_iota(jnp.int32, (1, RB), 1)
    valid = (i * RB + lanes) < N
    dv = jnp.where(valid, lax.rsqrt(deg), 0.0)
    ones = jnp.ones((1, D), jnp.float32)
    return lax.dot_general(dv, ones, (((0,), (0,)), ((), ())),
                           preferred_element_type=jnp.float32)


def _mm1_body(cnt_ref, x_ref, w_ref, g_ref):
    i = pl.program_id(0)
    dm = _dinv_mat(cnt_ref, i)
    xw = jnp.dot(x_ref[...], w_ref[...], preferred_element_type=jnp.float32)
    g_ref[...] = dm * xw


def _mm2_body(cnt_ref, acc_ref, g1_ref, w_ref, b_ref, g2_ref):
    i = pl.program_id(0)
    dm = _dinv_mat(cnt_ref, i)
    a = acc_ref[...]
    pre = dm * (a[0] + a[1] + g1_ref[...]) + b_ref[...]
    h = jnp.maximum(pre, 0.0)
    hw = jnp.dot(h, w_ref[...], preferred_element_type=jnp.float32)
    g2_ref[...] = dm * hw


def _fin_body(cnt_ref, acc_ref, g2_ref, b_ref, out_ref):
    i = pl.program_id(0)
    dm = _dinv_mat(cnt_ref, i)
    a = acc_ref[...]
    out_ref[...] = dm * (a[0] + a[1] + g2_ref[...]) + b_ref[...]


def kernel(x, edge_index, W1, b1, W2, b2):
    ei = edge_index.astype(jnp.int32)
    src, dst = ei[0], ei[1]
    e = src.shape[0]
    ep = NW * CH * K
    pad = ep - e
    padidx = N + (jnp.arange(pad, dtype=jnp.int32) % (NP - N))
    srcp = jnp.concatenate([src, padidx]).reshape(NW, CH, K)
    dstp = jnp.concatenate([dst, padidx]).reshape(NW, CH, K)
    dst2 = dst.reshape(NW, CHC, KC)
    xp = jnp.pad(x, ((0, NP - N), (0, 0)))
    ones_k = jnp.ones((KC,), jnp.float32)
    zn = jnp.zeros((RPT,), jnp.float32)
    znd = jnp.zeros((RPT, D), jnp.float32)

    mesh = plsc.VectorSubcoreMesh(core_axis_name="c", subcore_axis_name="s")

    cnt = pl.kernel(
        _count_body,
        out_type=jax.ShapeDtypeStruct((NC * NP,), jnp.float32),
        mesh=mesh,
        scratch_types=[
            pltpu.VMEM_SHARED((NP,), jnp.float32),
            pltpu.VMEM((CHC, KC), jnp.int32),
            pltpu.VMEM((KC,), jnp.float32),
            pltpu.SemaphoreType.DMA,
        ],
    )(dst2, ones_k, zn)
    cntr = cnt.reshape(NC, NRB, 1, RB)

    edge_call = pl.kernel(
        _edge_body,
        out_type=jax.ShapeDtypeStruct((NC * NP, D), jnp.float32),
        mesh=mesh,
        scratch_types=[
            pltpu.VMEM_SHARED((NP, D), jnp.float32),
            pltpu.VMEM((4, 2, K), jnp.int32),
            pltpu.VMEM((2, K, D), jnp.float32),
        ] + [pltpu.SemaphoreType.DMA] * 8,
    )

    g1 = pl.pallas_call(
        _mm1_body,
        grid=(NRB,),
        in_specs=[
            pl.BlockSpec((NC, 1, 1, RB), lambda i: (0, i, 0, 0)),
            pl.BlockSpec((RB, D), lambda i: (i, 0)),
            pl.BlockSpec((D, D), lambda i: (0, 0)),
        ],
        out_specs=pl.BlockSpec((RB, D), lambda i: (i, 0)),
        out_shape=jax.ShapeDtypeStruct((NP, D), jnp.float32),
    )(cntr, xp, W1)

    acc1 = edge_call(g1, srcp, dstp, znd).reshape(NC, NP, D)

    g2 = pl.pallas_call(
        _mm2_body,
        grid=(NRB,),
        in_specs=[
            pl.BlockSpec((NC, 1, 1, RB), lambda i: (0, i, 0, 0)),
            pl.BlockSpec((NC, RB, D), lambda i: (0, i, 0)),
            pl.BlockSpec((RB, D), lambda i: (i, 0)),
            pl.BlockSpec((D, D), lambda i: (0, 0)),
            pl.BlockSpec((1, D), lambda i: (0, 0)),
        ],
        out_specs=pl.BlockSpec((RB, D), lambda i: (i, 0)),
        out_shape=jax.ShapeDtypeStruct((NP, D), jnp.float32),
    )(cntr, acc1, g1, W2, b1.reshape(1, D))

    acc2 = edge_call(g2, srcp, dstp, znd).reshape(NC, NP, D)

    out = pl.pallas_call(
        _fin_body,
        grid=(NRB,),
        in_specs=[
            pl.BlockSpec((NC, 1, 1, RB), lambda i: (0, i, 0, 0)),
            pl.BlockSpec((NC, RB, D), lambda i: (0, i, 0)),
            pl.BlockSpec((RB, D), lambda i: (i, 0)),
            pl.BlockSpec((1, D), lambda i: (0, 0)),
        ],
        out_specs=pl.BlockSpec((RB, D), lambda i: (i, 0)),
        out_shape=jax.ShapeDtypeStruct((N, D), jnp.float32),
    )(cntr, acc2, g2, b2.reshape(1, D))

    return out


# speedup vs baseline: 34.9135x; 1.0137x over previous
import jax
import jax.numpy as jnp
from jax import lax
from jax.experimental import pallas as pl
from jax.experimental.pallas import tpu as pltpu
from jax.experimental.pallas import tpu_sc as plsc

N = 10000
D = 128
NC, NS = 2, 16
NW = NC * NS
NP = 10240
RPT = NP // NS
K = 128
CH = 80
KC = 80
CHC = 125
RB = 2048
NRB = NP // RB


def _count_body(e4_hbm, ones_hbm, zn_hbm, cnt_hbm, cnt_sh, dstb, ones_v,
                sem):
    c = lax.axis_index("c")
    s = lax.axis_index("s")
    w = c * NS + s
    pltpu.sync_copy(e4_hbm.at[1, w], dstb)
    pltpu.sync_copy(ones_hbm, ones_v)
    pltpu.sync_copy(zn_hbm, cnt_sh.at[pl.ds(s * RPT, RPT)])
    plsc.subcore_barrier()
    gf = 25

    def body(j, carry):
        for t in range(gf):
            pltpu.async_copy(ones_v, cnt_sh.at[dstb.at[j * gf + t]], sem,
                             add=True)
        for t in range(gf):
            pltpu.make_async_copy(ones_v, cnt_sh.at[dstb.at[0]], sem).wait()
        return carry

    lax.fori_loop(0, CHC // gf, body, 0)
    plsc.subcore_barrier()
    pltpu.sync_copy(cnt_sh.at[pl.ds(s * RPT, RPT)],
                    cnt_hbm.at[pl.ds(c * NP + s * RPT, RPT)])


def _edge_body(g_hbm, src_hbm, dst_hbm, znd_hbm, acc_hbm,
               acc_sh, idxb, rows, sg0, sg1, ss0, ss1, sx0, sx1, sx2, sx3):
    c = lax.axis_index("c")
    s = lax.axis_index("s")
    w = c * NS + s
    sg = [sg0, sg1]
    ss = [ss0, ss1]
    sx = [sx0, sx1, sx2, sx3]

    def xfire(i, q):
        pltpu.async_copy(src_hbm.at[w, i], idxb.at[q, 0], sx[q])
        pltpu.async_copy(dst_hbm.at[w, i], idxb.at[q, 1], sx[q])

    def xwait(q):
        pltpu.make_async_copy(src_hbm.at[0, 0], idxb.at[q, 0], sx[q]).wait()
        pltpu.make_async_copy(dst_hbm.at[0, 0], idxb.at[q, 1], sx[q]).wait()

    def gfire(b, q):
        pltpu.async_copy(g_hbm.at[idxb.at[q, 0]], rows.at[b], sg[b])

    def gwait(b):
        pltpu.make_async_copy(g_hbm.at[idxb.at[0, 0]], rows.at[b],
                              sg[b]).wait()

    def sfire(b, q):
        pltpu.async_copy(rows.at[b], acc_sh.at[idxb.at[q, 1]], ss[b],
                         add=True)

    def swait(b):
        pltpu.make_async_copy(rows.at[b], acc_sh.at[idxb.at[0, 1]],
                              ss[b]).wait()

    pltpu.sync_copy(znd_hbm, acc_sh.at[pl.ds(s * RPT, RPT)])
    xfire(0, 0)
    plsc.subcore_barrier()

    def body(j, carry):
        for u in range(4):
            i = j * 4 + u
            b = u % 2
            q = u
            qm = (u + 3) % 4

            if u >= 2:
                swait(b)
            else:
                @pl.when(j >= 1)
                def _():
                    swait(b)

            xwait(q)
            gfire(b, q)

            if u < 3:
                xfire(i + 1, u + 1)
            else:
                @pl.when(j <= CH // 4 - 2)
                def _():
                    xfire(i + 1, 0)

            if u >= 1:
                gwait(1 - b)
                sfire(1 - b, qm)
            else:
                @pl.when(j >= 1)
                def _():
                    gwait(1 - b)
                    sfire(1 - b, qm)
        return carry

    lax.fori_loop(0, CH // 4, body, 0)
    gwait(1)
    sfire(1, 3)
    swait(0)
    swait(1)
    plsc.subcore_barrier()
    pltpu.sync_copy(acc_sh.at[pl.ds(s * RPT, RPT)],
                    acc_hbm.at[pl.ds(c * NP + s * RPT, RPT)])


def _dinv_mat(cnt_ref, i):
    cz = cnt_ref[...]
    deg = (cz[0] + cz[1]).reshape(1, RB) + 1.0
    lanes = lax.broadcasted_iota(jnp.int32, (1, RB), 1)
    valid = (i * RB + lanes) < N
    dv = jnp.where(valid, lax.rsqrt(deg), 0.0)
    ones = jnp.ones((1, D), jnp.float32)
    return lax.dot_general(dv, ones, (((0,), (0,)), ((), ())),
                           preferred_element_type=jnp.float32)


def _mm1_body(cnt_ref, x_ref, w_ref, g_ref):
    i = pl.program_id(0)
    dm = _dinv_mat(cnt_ref, i)
    xw = jnp.dot(x_ref[...], w_ref[...], preferred_element_type=jnp.float32)
    g_ref[...] = dm * xw


def _mm2_body(cnt_ref, acc_ref, g1_ref, w_ref, b_ref, g2_ref):
    i = pl.program_id(0)
    dm = _dinv_mat(cnt_ref, i)
    a = acc_ref[...]
    pre = dm * (a[0] + a[1] + g1_ref[...]) + b_ref[...]
    h = jnp.maximum(pre, 0.0)
    hw = jnp.dot(h, w_ref[...], preferred_element_type=jnp.float32)
    g2_ref[...] = dm * hw


def _fin_body(cnt_ref, acc_ref, g2_ref, b_ref, out_ref):
    i = pl.program_id(0)
    dm = _dinv_mat(cnt_ref, i)
    a = acc_ref[...]
    out_ref[...] = dm * (a[0] + a[1] + g2_ref[...]) + b_ref[...]


def kernel(x, edge_index, W1, b1, W2, b2):
    ei = edge_index.astype(jnp.int32)
    src, dst = ei[0], ei[1]
    e = src.shape[0]
    ep = NW * CH * K
    pad = ep - e
    padidx = N + (jnp.arange(pad, dtype=jnp.int32) % (NP - N))
    srcp = jnp.concatenate([src, padidx]).reshape(NW, CH, K)
    dstp = jnp.concatenate([dst, padidx]).reshape(NW, CH, K)
    e4 = ei.reshape(2, NW, CHC, KC)
    xp = jnp.pad(x, ((0, NP - N), (0, 0)))
    ones_k = jnp.ones((KC,), jnp.float32)
    zn = jnp.zeros((RPT,), jnp.float32)
    znd = jnp.zeros((RPT, D), jnp.float32)

    mesh = plsc.VectorSubcoreMesh(core_axis_name="c", subcore_axis_name="s")

    cnt = pl.kernel(
        _count_body,
        out_type=jax.ShapeDtypeStruct((NC * NP,), jnp.float32),
        mesh=mesh,
        scratch_types=[
            pltpu.VMEM_SHARED((NP,), jnp.float32),
            pltpu.VMEM((CHC, KC), jnp.int32),
            pltpu.VMEM((KC,), jnp.float32),
            pltpu.SemaphoreType.DMA,
        ],
    )(e4, ones_k, zn)
    cntr = cnt.reshape(NC, NRB, 1, RB)

    edge_call = pl.kernel(
        _edge_body,
        out_type=jax.ShapeDtypeStruct((NC * NP, D), jnp.float32),
        mesh=mesh,
        scratch_types=[
            pltpu.VMEM_SHARED((NP, D), jnp.float32),
            pltpu.VMEM((4, 2, K), jnp.int32),
            pltpu.VMEM((2, K, D), jnp.float32),
        ] + [pltpu.SemaphoreType.DMA] * 8,
    )

    g1 = pl.pallas_call(
        _mm1_body,
        grid=(NRB,),
        in_specs=[
            pl.BlockSpec((NC, 1, 1, RB), lambda i: (0, i, 0, 0)),
            pl.BlockSpec((RB, D), lambda i: (i, 0)),
            pl.BlockSpec((D, D), lambda i: (0, 0)),
        ],
        out_specs=pl.BlockSpec((RB, D), lambda i: (i, 0)),
        out_shape=jax.ShapeDtypeStruct((NP, D), jnp.float32),
    )(cntr, xp, W1)

    acc1 = edge_call(g1, srcp, dstp, znd).reshape(NC, NP, D)

    g2 = pl.pallas_call(
        _mm2_body,
        grid=(NRB,),
        in_specs=[
            pl.BlockSpec((NC, 1, 1, RB), lambda i: (0, i, 0, 0)),
            pl.BlockSpec((NC, RB, D), lambda i: (0, i, 0)),
            pl.BlockSpec((RB, D), lambda i: (i, 0)),
            pl.BlockSpec((D, D), lambda i: (0, 0)),
            pl.BlockSpec((1, D), lambda i: (0, 0)),
        ],
        out_specs=pl.BlockSpec((RB, D), lambda i: (i, 0)),
        out_shape=jax.ShapeDtypeStruct((NP, D), jnp.float32),
    )(cntr, acc1, g1, W2, b1.reshape(1, D))

    acc2 = edge_call(g2, srcp, dstp, znd).reshape(NC, NP, D)

    out = pl.pallas_call(
        _fin_body,
        grid=(NRB,),
        in_specs=[
            pl.BlockSpec((NC, 1, 1, RB), lambda i: (0, i, 0, 0)),
            pl.BlockSpec((NC, RB, D), lambda i: (0, i, 0)),
            pl.BlockSpec((RB, D), lambda i: (i, 0)),
            pl.BlockSpec((1, D), lambda i: (0, 0)),
        ],
        out_specs=pl.BlockSpec((RB, D), lambda i: (i, 0)),
        out_shape=jax.ShapeDtypeStruct((N, D), jnp.float32),
    )(cntr, acc2, g2, b2.reshape(1, D))

    return out


# speedup vs baseline: 36.0447x; 1.0324x over previous
import jax
import jax.numpy as jnp
import numpy as np
from jax import lax
from jax.experimental import pallas as pl
from jax.experimental.pallas import tpu as pltpu
from jax.experimental.pallas import tpu_sc as plsc

N = 10000
D = 128
NC, NS = 2, 16
NW = NC * NS
NP = 10240
RPT = NP // NS
K = 128
KC = 80
CHC = 125
EPT = 10000
CHF = EPT // K
REM = EPT - CHF * K
RB = 2048
NRB = NP // RB


def _count_body(e4_hbm, ones_hbm, zn_hbm, cnt_hbm, cnt_sh, dstb, ones_v,
                sem):
    c = lax.axis_index("c")
    s = lax.axis_index("s")
    w = c * NS + s
    pltpu.sync_copy(e4_hbm.at[1, w], dstb)
    pltpu.sync_copy(ones_hbm, ones_v)
    pltpu.sync_copy(zn_hbm, cnt_sh.at[pl.ds(s * RPT, RPT)])
    plsc.subcore_barrier()
    gf = 25

    def body(j, carry):
        for t in range(gf):
            pltpu.async_copy(ones_v, cnt_sh.at[dstb.at[j * gf + t]], sem,
                             add=True)
        for t in range(gf):
            pltpu.make_async_copy(ones_v, cnt_sh.at[dstb.at[0]], sem).wait()
        return carry

    lax.fori_loop(0, CHC // gf, body, 0)
    plsc.subcore_barrier()
    pltpu.sync_copy(cnt_sh.at[pl.ds(s * RPT, RPT)],
                    cnt_hbm.at[pl.ds(c * NP + s * RPT, RPT)])


def _edge_body(g_hbm, e3_hbm, znd_hbm, acc_hbm,
               acc_sh, idxb, rows, sg0, sg1, ss0, ss1, sx0, sx1, sx2, sx3):
    c = lax.axis_index("c")
    s = lax.axis_index("s")
    w = c * NS + s
    sg = [sg0, sg1]
    ss = [ss0, ss1]
    sx = [sx0, sx1, sx2, sx3]

    def xfire(i, q):
        pltpu.async_copy(e3_hbm.at[0, w, pl.ds(i * K, K)], idxb.at[q, 0],
                         sx[q])
        pltpu.async_copy(e3_hbm.at[1, w, pl.ds(i * K, K)], idxb.at[q, 1],
                         sx[q])

    def xwait(q):
        pltpu.make_async_copy(e3_hbm.at[0, 0, pl.ds(0, K)], idxb.at[q, 0],
                              sx[q]).wait()
        pltpu.make_async_copy(e3_hbm.at[0, 0, pl.ds(0, K)], idxb.at[q, 1],
                              sx[q]).wait()

    def gfire(b, q):
        pltpu.async_copy(g_hbm.at[idxb.at[q, 0]], rows.at[b], sg[b])

    def gwait(b):
        pltpu.make_async_copy(g_hbm.at[idxb.at[0, 0]], rows.at[b],
                              sg[b]).wait()

    def sfire(b, q):
        pltpu.async_copy(rows.at[b], acc_sh.at[idxb.at[q, 1]], ss[b],
                         add=True)

    def swait(b):
        pltpu.make_async_copy(rows.at[b], acc_sh.at[idxb.at[0, 1]],
                              ss[b]).wait()

    pltpu.sync_copy(znd_hbm, acc_sh.at[pl.ds(s * RPT, RPT)])
    xfire(0, 0)
    plsc.subcore_barrier()

    def body(j, carry):
        for u in range(4):
            b = u % 2
            q = u
            qm = (u + 3) % 4

            if u >= 2:
                swait(b)
            else:
                @pl.when(j >= 1)
                def _():
                    swait(b)

            xwait(q)
            gfire(b, q)
            xfire(j * 4 + u + 1, (u + 1) % 4)

            if u >= 1:
                gwait(1 - b)
                sfire(1 - b, qm)
            else:
                @pl.when(j >= 1)
                def _():
                    gwait(1 - b)
                    sfire(1 - b, qm)
        return carry

    lax.fori_loop(0, (CHF - 2) // 4, body, 0)

    swait(0)
    xwait(0)
    gfire(0, 0)
    xfire(CHF - 1, 1)
    gwait(1)
    sfire(1, 3)

    swait(1)
    xwait(1)
    gfire(1, 1)
    for t in range((K - REM) // 16):
        vals = N + t * 16 + lax.iota(jnp.int32, 16)
        idxb[2, 0, pl.ds(REM + t * 16, 16)] = vals
        idxb[2, 1, pl.ds(REM + t * 16, 16)] = vals
    pltpu.async_copy(e3_hbm.at[0, w, pl.ds(CHF * K, REM)],
                     idxb.at[2, 0, pl.ds(0, REM)], sx[2])
    pltpu.async_copy(e3_hbm.at[1, w, pl.ds(CHF * K, REM)],
                     idxb.at[2, 1, pl.ds(0, REM)], sx[2])
    gwait(0)
    sfire(0, 0)

    swait(0)
    pltpu.make_async_copy(e3_hbm.at[0, 0, pl.ds(0, REM)],
                          idxb.at[2, 0, pl.ds(0, REM)], sx[2]).wait()
    pltpu.make_async_copy(e3_hbm.at[0, 0, pl.ds(0, REM)],
                          idxb.at[2, 1, pl.ds(0, REM)], sx[2]).wait()
    gfire(0, 2)
    gwait(1)
    sfire(1, 1)

    gwait(0)
    sfire(0, 2)
    swait(1)
    swait(0)
    plsc.subcore_barrier()
    pltpu.sync_copy(acc_sh.at[pl.ds(s * RPT, RPT)],
                    acc_hbm.at[pl.ds(c * NP + s * RPT, RPT)])


def _dinv_mat(cnt_ref, i):
    cz = cnt_ref[...]
    deg = (cz[0] + cz[1]).reshape(1, RB) + 1.0
    lanes = lax.broadcasted_iota(jnp.int32, (1, RB), 1)
    valid = (i * RB + lanes) < N
    dv = jnp.where(valid, lax.rsqrt(deg), 0.0)
    ones = jnp.ones((1, D), jnp.float32)
    return lax.dot_general(dv, ones, (((0,), (0,)), ((), ())),
                           preferred_element_type=jnp.float32)


def _mm1_body(cnt_ref, x_ref, w_ref, g_ref):
    i = pl.program_id(0)
    dm = _dinv_mat(cnt_ref, i)
    xw = jnp.dot(x_ref[...], w_ref[...], preferred_element_type=jnp.float32)
    g_ref[...] = dm * xw


def _mm2_body(cnt_ref, acc_ref, g1_ref, w_ref, b_ref, g2_ref):
    i = pl.program_id(0)
    dm = _dinv_mat(cnt_ref, i)
    a = acc_ref[...]
    pre = dm * (a[0] + a[1] + g1_ref[...]) + b_ref[...]
    h = jnp.maximum(pre, 0.0)
    hw = jnp.dot(h, w_ref[...], preferred_element_type=jnp.float32)
    g2_ref[...] = dm * hw


def _fin_body(cnt_ref, acc_ref, g2_ref, b_ref, out_ref):
    i = pl.program_id(0)
    dm = _dinv_mat(cnt_ref, i)
    a = acc_ref[...]
    out_ref[...] = dm * (a[0] + a[1] + g2_ref[...]) + b_ref[...]


def kernel(x, edge_index, W1, b1, W2, b2):
    ei = edge_index.astype(jnp.int32)
    e4 = ei.reshape(2, NW, CHC, KC)
    e3 = ei.reshape(2, NW, EPT)
    xp = jnp.pad(x, ((0, NP - N), (0, 0)))
    ones_k = np.ones((KC,), np.float32)
    zn = np.zeros((RPT,), np.float32)
    znd = np.zeros((RPT, D), np.float32)

    mesh = plsc.VectorSubcoreMesh(core_axis_name="c", subcore_axis_name="s")

    cnt = pl.kernel(
        _count_body,
        out_type=jax.ShapeDtypeStruct((NC * NP,), jnp.float32),
        mesh=mesh,
        scratch_types=[
            pltpu.VMEM_SHARED((NP,), jnp.float32),
            pltpu.VMEM((CHC, KC), jnp.int32),
            pltpu.VMEM((KC,), jnp.float32),
            pltpu.SemaphoreType.DMA,
        ],
    )(e4, ones_k, zn)
    cntr = cnt.reshape(NC, NRB, 1, RB)

    edge_call = pl.kernel(
        _edge_body,
        out_type=jax.ShapeDtypeStruct((NC * NP, D), jnp.float32),
        mesh=mesh,
        scratch_types=[
            pltpu.VMEM_SHARED((NP, D), jnp.float32),
            pltpu.VMEM((4, 2, K), jnp.int32),
            pltpu.VMEM((2, K, D), jnp.float32),
        ] + [pltpu.SemaphoreType.DMA] * 8,
    )

    g1 = pl.pallas_call(
        _mm1_body,
        grid=(NRB,),
        in_specs=[
            pl.BlockSpec((NC, 1, 1, RB), lambda i: (0, i, 0, 0)),
            pl.BlockSpec((RB, D), lambda i: (i, 0)),
            pl.BlockSpec((D, D), lambda i: (0, 0)),
        ],
        out_specs=pl.BlockSpec((RB, D), lambda i: (i, 0)),
        out_shape=jax.ShapeDtypeStruct((NP, D), jnp.float32),
    )(cntr, xp, W1)

    acc1 = edge_call(g1, e3, znd).reshape(NC, NP, D)

    g2 = pl.pallas_call(
        _mm2_body,
        grid=(NRB,),
        in_specs=[
            pl.BlockSpec((NC, 1, 1, RB), lambda i: (0, i, 0, 0)),
            pl.BlockSpec((NC, RB, D), lambda i: (0, i, 0)),
            pl.BlockSpec((RB, D), lambda i: (i, 0)),
            pl.BlockSpec((D, D), lambda i: (0, 0)),
            pl.BlockSpec((1, D), lambda i: (0, 0)),
        ],
        out_specs=pl.BlockSpec((RB, D), lambda i: (i, 0)),
        out_shape=jax.ShapeDtypeStruct((NP, D), jnp.float32),
    )(cntr, acc1, g1, W2, b1.reshape(1, D))

    acc2 = edge_call(g2, e3, znd).reshape(NC, NP, D)

    out = pl.pallas_call(
        _fin_body,
        grid=(NRB,),
        in_specs=[
            pl.BlockSpec((NC, 1, 1, RB), lambda i: (0, i, 0, 0)),
            pl.BlockSpec((NC, RB, D), lambda i: (0, i, 0)),
            pl.BlockSpec((RB, D), lambda i: (i, 0)),
            pl.BlockSpec((1, D), lambda i: (0, 0)),
        ],
        out_specs=pl.BlockSpec((RB, D), lambda i: (i, 0)),
        out_shape=jax.ShapeDtypeStruct((N, D), jnp.float32),
    )(cntr, acc2, g2, b2.reshape(1, D))

    return out


# speedup vs baseline: 36.1115x; 1.0019x over previous
import jax
import jax.numpy as jnp
import numpy as np
from jax import lax
from jax.experimental import pallas as pl
from jax.experimental.pallas import tpu as pltpu
from jax.experimental.pallas import tpu_sc as plsc

N = 10000
D = 128
NC, NS = 2, 16
NW = NC * NS
NP = 10240
RPT = NP // NS
K = 128
KC = 80
CHC = 125
EPT = 10000
CHF = EPT // K
REM = EPT - CHF * K
RB = 2048
NRB = NP // RB


def _count_body(e4_hbm, ones_hbm, zn_hbm, cnt_hbm, cnt_sh, dstb, ones_v,
                sem):
    c = lax.axis_index("c")
    s = lax.axis_index("s")
    w = c * NS + s
    pltpu.sync_copy(e4_hbm.at[1, w], dstb)
    pltpu.sync_copy(ones_hbm, ones_v)
    pltpu.sync_copy(zn_hbm, cnt_sh.at[pl.ds(s * RPT, RPT)])
    plsc.subcore_barrier()
    gf = 25

    def body(j, carry):
        for t in range(gf):
            pltpu.async_copy(ones_v, cnt_sh.at[dstb.at[j * gf + t]], sem,
                             add=True)
        for t in range(gf):
            pltpu.make_async_copy(ones_v, cnt_sh.at[dstb.at[0]], sem).wait()
        return carry

    lax.fori_loop(0, CHC // gf, body, 0)
    plsc.subcore_barrier()
    pltpu.sync_copy(cnt_sh.at[pl.ds(s * RPT, RPT)],
                    cnt_hbm.at[pl.ds(c * NP + s * RPT, RPT)])


def _edge_body(g_hbm, e3_hbm, znd_hbm, acc_hbm,
               acc_sh, idxb, rows, sg0, sg1, ss0, ss1, sx0, sx1, sx2, sx3):
    c = lax.axis_index("c")
    s = lax.axis_index("s")
    w = c * NS + s
    sg = [sg0, sg1]
    ss = [ss0, ss1]
    sx = [sx0, sx1, sx2, sx3]

    def xfire(i, q):
        pltpu.async_copy(e3_hbm.at[0, w, pl.ds(i * K, K)], idxb.at[q, 0],
                         sx[q])
        pltpu.async_copy(e3_hbm.at[1, w, pl.ds(i * K, K)], idxb.at[q, 1],
                         sx[q])

    def xwait(q):
        pltpu.make_async_copy(e3_hbm.at[0, 0, pl.ds(0, K)], idxb.at[q, 0],
                              sx[q]).wait()
        pltpu.make_async_copy(e3_hbm.at[0, 0, pl.ds(0, K)], idxb.at[q, 1],
                              sx[q]).wait()

    def gfire(b, q):
        pltpu.async_copy(g_hbm.at[idxb.at[q, 0, pl.ds(0, K // 2)]],
                         rows.at[b, pl.ds(0, K // 2)], sg[b])
        pltpu.async_copy(g_hbm.at[idxb.at[q, 0, pl.ds(K // 2, K // 2)]],
                         rows.at[b, pl.ds(K // 2, K // 2)], sg[b])

    def gwait(b):
        pltpu.make_async_copy(g_hbm.at[idxb.at[0, 0, pl.ds(0, K // 2)]],
                              rows.at[b, pl.ds(0, K // 2)], sg[b]).wait()
        pltpu.make_async_copy(g_hbm.at[idxb.at[0, 0, pl.ds(0, K // 2)]],
                              rows.at[b, pl.ds(K // 2, K // 2)],
                              sg[b]).wait()

    def sfire(b, q):
        pltpu.async_copy(rows.at[b], acc_sh.at[idxb.at[q, 1]], ss[b],
                         add=True)

    def swait(b):
        pltpu.make_async_copy(rows.at[b], acc_sh.at[idxb.at[0, 1]],
                              ss[b]).wait()

    pltpu.sync_copy(znd_hbm, acc_sh.at[pl.ds(s * RPT, RPT)])
    xfire(0, 0)
    plsc.subcore_barrier()

    def body(j, carry):
        for u in range(4):
            b = u % 2
            q = u
            qm = (u + 3) % 4

            if u >= 2:
                swait(b)
            else:
                @pl.when(j >= 1)
                def _():
                    swait(b)

            xwait(q)
            gfire(b, q)
            xfire(j * 4 + u + 1, (u + 1) % 4)

            if u >= 1:
                gwait(1 - b)
                sfire(1 - b, qm)
            else:
                @pl.when(j >= 1)
                def _():
                    gwait(1 - b)
                    sfire(1 - b, qm)
        return carry

    lax.fori_loop(0, (CHF - 2) // 4, body, 0)

    swait(0)
    xwait(0)
    gfire(0, 0)
    xfire(CHF - 1, 1)
    gwait(1)
    sfire(1, 3)

    swait(1)
    xwait(1)
    gfire(1, 1)
    for t in range((K - REM) // 16):
        vals = N + t * 16 + lax.iota(jnp.int32, 16)
        idxb[2, 0, pl.ds(REM + t * 16, 16)] = vals
        idxb[2, 1, pl.ds(REM + t * 16, 16)] = vals
    pltpu.async_copy(e3_hbm.at[0, w, pl.ds(CHF * K, REM)],
                     idxb.at[2, 0, pl.ds(0, REM)], sx[2])
    pltpu.async_copy(e3_hbm.at[1, w, pl.ds(CHF * K, REM)],
                     idxb.at[2, 1, pl.ds(0, REM)], sx[2])
    gwait(0)
    sfire(0, 0)

    swait(0)
    pltpu.make_async_copy(e3_hbm.at[0, 0, pl.ds(0, REM)],
                          idxb.at[2, 0, pl.ds(0, REM)], sx[2]).wait()
    pltpu.make_async_copy(e3_hbm.at[0, 0, pl.ds(0, REM)],
                          idxb.at[2, 1, pl.ds(0, REM)], sx[2]).wait()
    gfire(0, 2)
    gwait(1)
    sfire(1, 1)

    gwait(0)
    sfire(0, 2)
    swait(1)
    swait(0)
    plsc.subcore_barrier()
    pltpu.sync_copy(acc_sh.at[pl.ds(s * RPT, RPT)],
                    acc_hbm.at[pl.ds(c * NP + s * RPT, RPT)])


def _dinv_mat(cnt_ref, i):
    cz = cnt_ref[...]
    deg = (cz[0] + cz[1]).reshape(1, RB) + 1.0
    lanes = lax.broadcasted_iota(jnp.int32, (1, RB), 1)
    valid = (i * RB + lanes) < N
    dv = jnp.where(valid, lax.rsqrt(deg), 0.0)
    ones = jnp.ones((1, D), jnp.float32)
    return lax.dot_general(dv, ones, (((0,), (0,)), ((), ())),
                           preferred_element_type=jnp.float32)


def _mm1_body(cnt_ref, x_ref, w_ref, g_ref):
    i = pl.program_id(0)
    dm = _dinv_mat(cnt_ref, i)
    xw = jnp.dot(x_ref[...], w_ref[...], preferred_element_type=jnp.float32)
    g_ref[...] = dm * xw


def _mm2_body(cnt_ref, acc_ref, g1_ref, w_ref, b_ref, g2_ref):
    i = pl.program_id(0)
    dm = _dinv_mat(cnt_ref, i)
    a = acc_ref[...]
    pre = dm * (a[0] + a[1] + g1_ref[...]) + b_ref[...]
    h = jnp.maximum(pre, 0.0)
    hw = jnp.dot(h, w_ref[...], preferred_element_type=jnp.float32)
    g2_ref[...] = dm * hw


def _fin_body(cnt_ref, acc_ref, g2_ref, b_ref, out_ref):
    i = pl.program_id(0)
    dm = _dinv_mat(cnt_ref, i)
    a = acc_ref[...]
    out_ref[...] = dm * (a[0] + a[1] + g2_ref[...]) + b_ref[...]


def kernel(x, edge_index, W1, b1, W2, b2):
    ei = edge_index.astype(jnp.int32)
    e4 = ei.reshape(2, NW, CHC, KC)
    e3 = ei.reshape(2, NW, EPT)
    xp = jnp.pad(x, ((0, NP - N), (0, 0)))
    ones_k = np.ones((KC,), np.float32)
    zn = np.zeros((RPT,), np.float32)
    znd = np.zeros((RPT, D), np.float32)

    mesh = plsc.VectorSubcoreMesh(core_axis_name="c", subcore_axis_name="s")

    cnt = pl.kernel(
        _count_body,
        out_type=jax.ShapeDtypeStruct((NC * NP,), jnp.float32),
        mesh=mesh,
        scratch_types=[
            pltpu.VMEM_SHARED((NP,), jnp.float32),
            pltpu.VMEM((CHC, KC), jnp.int32),
            pltpu.VMEM((KC,), jnp.float32),
            pltpu.SemaphoreType.DMA,
        ],
    )(e4, ones_k, zn)
    cntr = cnt.reshape(NC, NRB, 1, RB)

    edge_call = pl.kernel(
        _edge_body,
        out_type=jax.ShapeDtypeStruct((NC * NP, D), jnp.float32),
        mesh=mesh,
        scratch_types=[
            pltpu.VMEM_SHARED((NP, D), jnp.float32),
            pltpu.VMEM((4, 2, K), jnp.int32),
            pltpu.VMEM((2, K, D), jnp.float32),
        ] + [pltpu.SemaphoreType.DMA] * 8,
    )

    g1 = pl.pallas_call(
        _mm1_body,
        grid=(NRB,),
        in_specs=[
            pl.BlockSpec((NC, 1, 1, RB), lambda i: (0, i, 0, 0)),
            pl.BlockSpec((RB, D), lambda i: (i, 0)),
            pl.BlockSpec((D, D), lambda i: (0, 0)),
        ],
        out_specs=pl.BlockSpec((RB, D), lambda i: (i, 0)),
        out_shape=jax.ShapeDtypeStruct((NP, D), jnp.float32),
    )(cntr, xp, W1)

    acc1 = edge_call(g1, e3, znd).reshape(NC, NP, D)

    g2 = pl.pallas_call(
        _mm2_body,
        grid=(NRB,),
        in_specs=[
            pl.BlockSpec((NC, 1, 1, RB), lambda i: (0, i, 0, 0)),
            pl.BlockSpec((NC, RB, D), lambda i: (0, i, 0)),
            pl.BlockSpec((RB, D), lambda i: (i, 0)),
            pl.BlockSpec((D, D), lambda i: (0, 0)),
            pl.BlockSpec((1, D), lambda i: (0, 0)),
        ],
        out_specs=pl.BlockSpec((RB, D), lambda i: (i, 0)),
        out_shape=jax.ShapeDtypeStruct((NP, D), jnp.float32),
    )(cntr, acc1, g1, W2, b1.reshape(1, D))

    acc2 = edge_call(g2, e3, znd).reshape(NC, NP, D)

    out = pl.pallas_call(
        _fin_body,
        grid=(NRB,),
        in_specs=[
            pl.BlockSpec((NC, 1, 1, RB), lambda i: (0, i, 0, 0)),
            pl.BlockSpec((NC, RB, D), lambda i: (0, i, 0)),
            pl.BlockSpec((RB, D), lambda i: (i, 0)),
            pl.BlockSpec((1, D), lambda i: (0, 0)),
        ],
        out_specs=pl.BlockSpec((RB, D), lambda i: (i, 0)),
        out_shape=jax.ShapeDtypeStruct((N, D), jnp.float32),
    )(cntr, acc2, g2, b2.reshape(1, D))

    return out


# speedup vs baseline: 36.1207x; 1.0003x over previous
import jax
import jax.numpy as jnp
import numpy as np
from jax import lax
from jax.experimental import pallas as pl
from jax.experimental.pallas import tpu as pltpu
from jax.experimental.pallas import tpu_sc as plsc

N = 10000
D = 128
NC, NS = 2, 16
NW = NC * NS
NP = 10240
RPT = NP // NS
K = 128
KC = 80
CHC = 125
EPT = 10000
CHF = EPT // K
REM = EPT - CHF * K
RB = 2048
NRB = NP // RB


def _count_body(e4_hbm, ones_hbm, zn_hbm, cnt_hbm, cnt_sh, dstb, ones_v,
                sem):
    c = lax.axis_index("c")
    s = lax.axis_index("s")
    w = c * NS + s
    pltpu.sync_copy(e4_hbm.at[1, w], dstb)
    pltpu.sync_copy(ones_hbm, ones_v)
    pltpu.sync_copy(zn_hbm, cnt_sh.at[pl.ds(s * RPT, RPT)])
    plsc.subcore_barrier()
    gf = 25

    def body(j, carry):
        for t in range(gf):
            pltpu.async_copy(ones_v, cnt_sh.at[dstb.at[j * gf + t]], sem,
                             add=True)
        for t in range(gf):
            pltpu.make_async_copy(ones_v, cnt_sh.at[dstb.at[0]], sem).wait()
        return carry

    lax.fori_loop(0, CHC // gf, body, 0)
    plsc.subcore_barrier()
    pltpu.sync_copy(cnt_sh.at[pl.ds(s * RPT, RPT)],
                    cnt_hbm.at[pl.ds(c * NP + s * RPT, RPT)])


def _edge_body(g_hbm, e3_hbm, znd_hbm, acc_hbm,
               acc_sh, idxb, rows, sg0, sg1, ss0, ss1, sx0, sx1, sx2, sx3):
    c = lax.axis_index("c")
    s = lax.axis_index("s")
    w = c * NS + s
    sg = [sg0, sg1]
    ss = [ss0, ss1]
    sx = [sx0, sx1, sx2, sx3]

    def xfire(i, q):
        pltpu.async_copy(e3_hbm.at[0, w, pl.ds(i * K, K)], idxb.at[q, 0],
                         sx[q])
        pltpu.async_copy(e3_hbm.at[1, w, pl.ds(i * K, K)], idxb.at[q, 1],
                         sx[q])

    def xwait(q):
        pltpu.make_async_copy(e3_hbm.at[0, 0, pl.ds(0, K)], idxb.at[q, 0],
                              sx[q]).wait()
        pltpu.make_async_copy(e3_hbm.at[0, 0, pl.ds(0, K)], idxb.at[q, 1],
                              sx[q]).wait()

    def gfire(b, q):
        pltpu.async_copy(g_hbm.at[idxb.at[q, 0]], rows.at[b], sg[b])

    def gwait(b):
        pltpu.make_async_copy(g_hbm.at[idxb.at[0, 0]], rows.at[b],
                              sg[b]).wait()

    def sfire(b, q):
        pltpu.async_copy(rows.at[b], acc_sh.at[idxb.at[q, 1]], ss[b],
                         add=True)

    def swait(b):
        pltpu.make_async_copy(rows.at[b], acc_sh.at[idxb.at[0, 1]],
                              ss[b]).wait()

    pltpu.sync_copy(znd_hbm, acc_sh.at[pl.ds(s * RPT, RPT)])
    xfire(0, 0)
    plsc.subcore_barrier()

    def body(j, carry):
        for u in range(4):
            b = u % 2
            q = u
            qm = (u + 3) % 4

            if u >= 2:
                swait(b)
            else:
                @pl.when(j >= 1)
                def _():
                    swait(b)

            xwait(q)
            gfire(b, q)
            xfire(j * 4 + u + 1, (u + 1) % 4)

            if u >= 1:
                gwait(1 - b)
                sfire(1 - b, qm)
            else:
                @pl.when(j >= 1)
                def _():
                    gwait(1 - b)
                    sfire(1 - b, qm)
        return carry

    lax.fori_loop(0, (CHF - 2) // 4, body, 0)

    swait(0)
    xwait(0)
    gfire(0, 0)
    xfire(CHF - 1, 1)
    gwait(1)
    sfire(1, 3)

    swait(1)
    xwait(1)
    gfire(1, 1)
    for t in range((K - REM) // 16):
        vals = N + t * 16 + lax.iota(jnp.int32, 16)
        idxb[2, 0, pl.ds(REM + t * 16, 16)] = vals
        idxb[2, 1, pl.ds(REM + t * 16, 16)] = vals
    pltpu.async_copy(e3_hbm.at[0, w, pl.ds(CHF * K, REM)],
                     idxb.at[2, 0, pl.ds(0, REM)], sx[2])
    pltpu.async_copy(e3_hbm.at[1, w, pl.ds(CHF * K, REM)],
                     idxb.at[2, 1, pl.ds(0, REM)], sx[2])
    gwait(0)
    sfire(0, 0)

    swait(0)
    pltpu.make_async_copy(e3_hbm.at[0, 0, pl.ds(0, REM)],
                          idxb.at[2, 0, pl.ds(0, REM)], sx[2]).wait()
    pltpu.make_async_copy(e3_hbm.at[0, 0, pl.ds(0, REM)],
                          idxb.at[2, 1, pl.ds(0, REM)], sx[2]).wait()
    gfire(0, 2)
    gwait(1)
    sfire(1, 1)

    gwait(0)
    sfire(0, 2)
    swait(1)
    swait(0)
    plsc.subcore_barrier()
    pltpu.sync_copy(acc_sh.at[pl.ds(s * RPT, RPT)],
                    acc_hbm.at[pl.ds(c * NP + s * RPT, RPT)])


def _dinv_mat(cnt_ref, i):
    cz = cnt_ref[...]
    deg = (cz[0] + cz[1]).reshape(1, RB) + 1.0
    lanes = lax.broadcasted_iota(jnp.int32, (1, RB), 1)
    valid = (i * RB + lanes) < N
    dv = jnp.where(valid, lax.rsqrt(deg), 0.0)
    ones = jnp.ones((1, D), jnp.float32)
    return lax.dot_general(dv, ones, (((0,), (0,)), ((), ())),
                           preferred_element_type=jnp.float32)


def _mm1_body(cnt_ref, x_ref, w_ref, g_ref):
    i = pl.program_id(0)
    dm = _dinv_mat(cnt_ref, i)
    xw = jnp.dot(x_ref[...], w_ref[...], preferred_element_type=jnp.float32)
    g_ref[...] = dm * xw


def _mm2_body(cnt_ref, acc_ref, g1_ref, w_ref, b_ref, g2_ref):
    i = pl.program_id(0)
    dm = _dinv_mat(cnt_ref, i)
    a = acc_ref[...]
    pre = dm * (a[0] + a[1] + g1_ref[...]) + b_ref[...]
    h = jnp.maximum(pre, 0.0)
    hw = jnp.dot(h, w_ref[...], preferred_element_type=jnp.float32)
    g2_ref[...] = dm * hw


def _fin_body(cnt_ref, acc_ref, g2_ref, b_ref, out_ref):
    i = pl.program_id(0)
    dm = _dinv_mat(cnt_ref, i)
    a = acc_ref[...]
    out_ref[...] = dm * (a[0] + a[1] + g2_ref[...]) + b_ref[...]


def kernel(x, edge_index, W1, b1, W2, b2):
    ei = edge_index.astype(jnp.int32)
    e4 = ei.reshape(2, NW, CHC, KC)
    e3 = ei.reshape(2, NW, EPT)
    xp = jnp.pad(x, ((0, NP - N), (0, 0)))
    ones_k = np.ones((KC,), np.float32)
    zn = np.zeros((RPT,), np.float32)
    znd = np.zeros((RPT, D), np.float32)

    mesh = plsc.VectorSubcoreMesh(core_axis_name="c", subcore_axis_name="s")

    cnt = pl.kernel(
        _count_body,
        out_type=jax.ShapeDtypeStruct((NC * NP,), jnp.float32),
        mesh=mesh,
        scratch_types=[
            pltpu.VMEM_SHARED((NP,), jnp.float32),
            pltpu.VMEM((CHC, KC), jnp.int32),
            pltpu.VMEM((KC,), jnp.float32),
            pltpu.SemaphoreType.DMA,
        ],
    )(e4, ones_k, zn)
    cntr = cnt.reshape(NC, NRB, 1, RB)

    edge_call = pl.kernel(
        _edge_body,
        out_type=jax.ShapeDtypeStruct((NC * NP, D), jnp.float32),
        mesh=mesh,
        scratch_types=[
            pltpu.VMEM_SHARED((NP, D), jnp.float32),
            pltpu.VMEM((4, 2, K), jnp.int32),
            pltpu.VMEM((2, K, D), jnp.float32),
        ] + [pltpu.SemaphoreType.DMA] * 8,
    )

    g1 = pl.pallas_call(
        _mm1_body,
        grid=(NRB,),
        in_specs=[
            pl.BlockSpec((NC, 1, 1, RB), lambda i: (0, i, 0, 0)),
            pl.BlockSpec((RB, D), lambda i: (i, 0)),
            pl.BlockSpec((D, D), lambda i: (0, 0)),
        ],
        out_specs=pl.BlockSpec((RB, D), lambda i: (i, 0)),
        out_shape=jax.ShapeDtypeStruct((NP, D), jnp.float32),
    )(cntr, xp, W1)

    acc1 = edge_call(g1, e3, znd).reshape(NC, NP, D)

    g2 = pl.pallas_call(
        _mm2_body,
        grid=(NRB,),
        in_specs=[
            pl.BlockSpec((NC, 1, 1, RB), lambda i: (0, i, 0, 0)),
            pl.BlockSpec((NC, RB, D), lambda i: (0, i, 0)),
            pl.BlockSpec((RB, D), lambda i: (i, 0)),
            pl.BlockSpec((D, D), lambda i: (0, 0)),
            pl.BlockSpec((1, D), lambda i: (0, 0)),
        ],
        out_specs=pl.BlockSpec((RB, D), lambda i: (i, 0)),
        out_shape=jax.ShapeDtypeStruct((NP, D), jnp.float32),
    )(cntr, acc1, g1, W2, b1.reshape(1, D))

    acc2 = edge_call(g2, e3, znd).reshape(NC, NP, D)

    out = pl.pallas_call(
        _fin_body,
        grid=(NRB,),
        in_specs=[
            pl.BlockSpec((NC, 1, 1, RB), lambda i: (0, i, 0, 0)),
            pl.BlockSpec((NC, RB, D), lambda i: (0, i, 0)),
            pl.BlockSpec((RB, D), lambda i: (i, 0)),
            pl.BlockSpec((1, D), lambda i: (0, 0)),
        ],
        out_specs=pl.BlockSpec((RB, D), lambda i: (i, 0)),
        out_shape=jax.ShapeDtypeStruct((N, D), jnp.float32),
    )(cntr, acc2, g2, b2.reshape(1, D))

    return out


# speedup vs baseline: 36.1220x; 1.0000x over previous
import jax
import jax.numpy as jnp
import numpy as np
from jax import lax
from jax.experimental import pallas as pl
from jax.experimental.pallas import tpu as pltpu
from jax.experimental.pallas import tpu_sc as plsc

N = 10000
D = 128
NC, NS = 2, 16
NW = NC * NS
NP = 10240
RPT = NP // NS
K = 128
KC = 80
CHC = 125
EPT = 10000
CHF = EPT // K
REM = EPT - CHF * K
RB = 2048
NRB = NP // RB


def _count_body(e4_hbm, ones_hbm, zn_hbm, cnt_hbm, cnt_sh, dstb, ones_v,
                sem):
    c = lax.axis_index("c")
    s = lax.axis_index("s")
    w = c * NS + s
    pltpu.sync_copy(e4_hbm.at[1, w], dstb)
    pltpu.sync_copy(ones_hbm, ones_v)
    pltpu.sync_copy(zn_hbm, cnt_sh.at[pl.ds(s * RPT, RPT)])
    plsc.subcore_barrier()
    gf = 25

    def body(j, carry):
        for t in range(gf):
            pltpu.async_copy(ones_v, cnt_sh.at[dstb.at[j * gf + t]], sem,
                             add=True)
        for t in range(gf):
            pltpu.make_async_copy(ones_v, cnt_sh.at[dstb.at[0]], sem).wait()
        return carry

    lax.fori_loop(0, CHC // gf, body, 0)
    plsc.subcore_barrier()
    pltpu.sync_copy(cnt_sh.at[pl.ds(s * RPT, RPT)],
                    cnt_hbm.at[pl.ds(c * NP + s * RPT, RPT)])


def _edge_body(g_hbm, e3_hbm, znd_hbm, acc_hbm,
               acc_sh, idxb, rows, sg0, sg1, ss0, ss1, sx0, sx1, sx2, sx3):
    c = lax.axis_index("c")
    s = lax.axis_index("s")
    w = c * NS + s
    sg = [sg0, sg1]
    ss = [ss0, ss1]
    sx = [sx0, sx1, sx2, sx3]

    def xfire(i, q):
        pltpu.async_copy(e3_hbm.at[pl.ds(w * EPT + i * K, K)], idxb.at[q, 0],
                         sx[q])
        pltpu.async_copy(e3_hbm.at[pl.ds(NW * EPT + w * EPT + i * K, K)],
                         idxb.at[q, 1], sx[q])

    def xwait(q):
        pltpu.make_async_copy(e3_hbm.at[pl.ds(0, K)], idxb.at[q, 0],
                              sx[q]).wait()
        pltpu.make_async_copy(e3_hbm.at[pl.ds(0, K)], idxb.at[q, 1],
                              sx[q]).wait()

    def gfire(b, q):
        pltpu.async_copy(g_hbm.at[idxb.at[q, 0]], rows.at[b], sg[b])

    def gwait(b):
        pltpu.make_async_copy(g_hbm.at[idxb.at[0, 0]], rows.at[b],
                              sg[b]).wait()

    def sfire(b, q):
        pltpu.async_copy(rows.at[b], acc_sh.at[idxb.at[q, 1]], ss[b],
                         add=True)

    def swait(b):
        pltpu.make_async_copy(rows.at[b], acc_sh.at[idxb.at[0, 1]],
                              ss[b]).wait()

    pltpu.sync_copy(znd_hbm, acc_sh.at[pl.ds(s * RPT, RPT)])
    xfire(0, 0)
    plsc.subcore_barrier()

    def body(j, carry):
        for u in range(4):
            b = u % 2
            q = u
            qm = (u + 3) % 4

            if u >= 2:
                swait(b)
            else:
                @pl.when(j >= 1)
                def _():
                    swait(b)

            xwait(q)
            gfire(b, q)
            xfire(j * 4 + u + 1, (u + 1) % 4)

            if u >= 1:
                gwait(1 - b)
                sfire(1 - b, qm)
            else:
                @pl.when(j >= 1)
                def _():
                    gwait(1 - b)
                    sfire(1 - b, qm)
        return carry

    lax.fori_loop(0, (CHF - 2) // 4, body, 0)

    swait(0)
    xwait(0)
    gfire(0, 0)
    xfire(CHF - 1, 1)
    gwait(1)
    sfire(1, 3)

    swait(1)
    xwait(1)
    gfire(1, 1)
    for t in range((K - REM) // 16):
        vals = N + t * 16 + lax.iota(jnp.int32, 16)
        idxb[2, 0, pl.ds(REM + t * 16, 16)] = vals
        idxb[2, 1, pl.ds(REM + t * 16, 16)] = vals
    pltpu.async_copy(e3_hbm.at[pl.ds(w * EPT + CHF * K, REM)],
                     idxb.at[2, 0, pl.ds(0, REM)], sx[2])
    pltpu.async_copy(e3_hbm.at[pl.ds(NW * EPT + w * EPT + CHF * K, REM)],
                     idxb.at[2, 1, pl.ds(0, REM)], sx[2])
    gwait(0)
    sfire(0, 0)

    swait(0)
    pltpu.make_async_copy(e3_hbm.at[pl.ds(0, REM)],
                          idxb.at[2, 0, pl.ds(0, REM)], sx[2]).wait()
    pltpu.make_async_copy(e3_hbm.at[pl.ds(0, REM)],
                          idxb.at[2, 1, pl.ds(0, REM)], sx[2]).wait()
    gfire(0, 2)
    gwait(1)
    sfire(1, 1)

    gwait(0)
    sfire(0, 2)
    swait(1)
    swait(0)
    plsc.subcore_barrier()
    pltpu.sync_copy(acc_sh.at[pl.ds(s * RPT, RPT)],
                    acc_hbm.at[pl.ds(c * NP + s * RPT, RPT)])


def _dinv_mat(cnt_ref, i):
    cz = cnt_ref[...]
    deg = (cz[0] + cz[1]).reshape(1, RB) + 1.0
    lanes = lax.broadcasted_iota(jnp.int32, (1, RB), 1)
    valid = (i * RB + lanes) < N
    dv = jnp.where(valid, lax.rsqrt(deg), 0.0)
    ones = jnp.ones((1, D), jnp.float32)
    return lax.dot_general(dv, ones, (((0,), (0,)), ((), ())),
                           preferred_element_type=jnp.float32)


def _mm1_body(cnt_ref, x_ref, w_ref, g_ref):
    i = pl.program_id(0)
    dm = _dinv_mat(cnt_ref, i)
    xw = jnp.dot(x_ref[...], w_ref[...], preferred_element_type=jnp.float32)
    g_ref[...] = dm * xw


def _mm2_body(cnt_ref, acc_ref, g1_ref, w_ref, b_ref, g2_ref):
    i = pl.program_id(0)
    dm = _dinv_mat(cnt_ref, i)
    a = acc_ref[...]
    pre = dm * (a[0] + a[1] + g1_ref[...]) + b_ref[...]
    h = jnp.maximum(pre, 0.0)
    hw = jnp.dot(h, w_ref[...], preferred_element_type=jnp.float32)
    g2_ref[...] = dm * hw


def _fin_body(cnt_ref, acc_ref, g2_ref, b_ref, out_ref):
    i = pl.program_id(0)
    dm = _dinv_mat(cnt_ref, i)
    a = acc_ref[...]
    out_ref[...] = dm * (a[0] + a[1] + g2_ref[...]) + b_ref[...]


def kernel(x, edge_index, W1, b1, W2, b2):
    ei = edge_index.astype(jnp.int32)
    e4 = ei.reshape(2, NW, CHC, KC)
    e3 = ei.reshape(2 * NW * EPT)
    xp = jnp.pad(x, ((0, NP - N), (0, 0)))
    ones_k = np.ones((KC,), np.float32)
    zn = np.zeros((RPT,), np.float32)
    znd = np.zeros((RPT, D), np.float32)

    mesh = plsc.VectorSubcoreMesh(core_axis_name="c", subcore_axis_name="s")

    cnt = pl.kernel(
        _count_body,
        out_type=jax.ShapeDtypeStruct((NC * NP,), jnp.float32),
        mesh=mesh,
        scratch_types=[
            pltpu.VMEM_SHARED((NP,), jnp.float32),
            pltpu.VMEM((CHC, KC), jnp.int32),
            pltpu.VMEM((KC,), jnp.float32),
            pltpu.SemaphoreType.DMA,
        ],
    )(e4, ones_k, zn)
    cntr = cnt.reshape(NC, NRB, 1, RB)

    edge_call = pl.kernel(
        _edge_body,
        out_type=jax.ShapeDtypeStruct((NC * NP, D), jnp.float32),
        mesh=mesh,
        scratch_types=[
            pltpu.VMEM_SHARED((NP, D), jnp.float32),
            pltpu.VMEM((4, 2, K), jnp.int32),
            pltpu.VMEM((2, K, D), jnp.float32),
        ] + [pltpu.SemaphoreType.DMA] * 8,
    )

    g1 = pl.pallas_call(
        _mm1_body,
        grid=(NRB,),
        in_specs=[
            pl.BlockSpec((NC, 1, 1, RB), lambda i: (0, i, 0, 0)),
            pl.BlockSpec((RB, D), lambda i: (i, 0)),
            pl.BlockSpec((D, D), lambda i: (0, 0)),
        ],
        out_specs=pl.BlockSpec((RB, D), lambda i: (i, 0)),
        out_shape=jax.ShapeDtypeStruct((NP, D), jnp.float32),
    )(cntr, xp, W1)

    acc1 = edge_call(g1, e3, znd).reshape(NC, NP, D)

    g2 = pl.pallas_call(
        _mm2_body,
        grid=(NRB,),
        in_specs=[
            pl.BlockSpec((NC, 1, 1, RB), lambda i: (0, i, 0, 0)),
            pl.BlockSpec((NC, RB, D), lambda i: (0, i, 0)),
            pl.BlockSpec((RB, D), lambda i: (i, 0)),
            pl.BlockSpec((D, D), lambda i: (0, 0)),
            pl.BlockSpec((1, D), lambda i: (0, 0)),
        ],
        out_specs=pl.BlockSpec((RB, D), lambda i: (i, 0)),
        out_shape=jax.ShapeDtypeStruct((NP, D), jnp.float32),
    )(cntr, acc1, g1, W2, b1.reshape(1, D))

    acc2 = edge_call(g2, e3, znd).reshape(NC, NP, D)

    out = pl.pallas_call(
        _fin_body,
        grid=(NRB,),
        in_specs=[
            pl.BlockSpec((NC, 1, 1, RB), lambda i: (0, i, 0, 0)),
            pl.BlockSpec((NC, RB, D), lambda i: (0, i, 0)),
            pl.BlockSpec((RB, D), lambda i: (i, 0)),
            pl.BlockSpec((1, D), lambda i: (0, 0)),
        ],
        out_specs=pl.BlockSpec((RB, D), lambda i: (i, 0)),
        out_shape=jax.ShapeDtypeStruct((N, D), jnp.float32),
    )(cntr, acc2, g2, b2.reshape(1, D))

    return out


# speedup vs baseline: 36.2504x; 1.0036x over previous
import jax
import jax.numpy as jnp
import numpy as np
from jax import lax
from jax.experimental import pallas as pl
from jax.experimental.pallas import tpu as pltpu
from jax.experimental.pallas import tpu_sc as plsc

N = 10000
D = 128
NC, NS = 2, 16
NW = NC * NS
NP = 10240
RPT = NP // NS
K = 128
KC = 80
CHC = 125
EPT = 10000
CHF = EPT // K
REM = EPT - CHF * K
RB = 2048
NRB = NP // RB


def _count_body(e3_hbm, ones_hbm, zn_hbm, cnt_hbm, cnt_sh, dstb, ones_v,
                sem):
    c = lax.axis_index("c")
    s = lax.axis_index("s")
    w = c * NS + s
    gf = 25
    base = NW * EPT + w * EPT

    def pre(j, carry):
        for t in range(gf):
            pltpu.async_copy(e3_hbm.at[pl.ds(base + (j * gf + t) * KC, KC)],
                             dstb.at[j * gf + t], sem)
        for t in range(gf):
            pltpu.make_async_copy(e3_hbm.at[pl.ds(0, KC)], dstb.at[0],
                                  sem).wait()
        return carry

    lax.fori_loop(0, CHC // gf, pre, 0)
    pltpu.sync_copy(ones_hbm, ones_v)
    pltpu.sync_copy(zn_hbm, cnt_sh.at[pl.ds(s * RPT, RPT)])
    plsc.subcore_barrier()

    def body(j, carry):
        for t in range(gf):
            pltpu.async_copy(ones_v, cnt_sh.at[dstb.at[j * gf + t]], sem,
                             add=True)
        for t in range(gf):
            pltpu.make_async_copy(ones_v, cnt_sh.at[dstb.at[0]], sem).wait()
        return carry

    lax.fori_loop(0, CHC // gf, body, 0)
    plsc.subcore_barrier()
    pltpu.sync_copy(cnt_sh.at[pl.ds(s * RPT, RPT)],
                    cnt_hbm.at[pl.ds(c * NP + s * RPT, RPT)])


def _edge_body(g_hbm, e3_hbm, znd_hbm, acc_hbm,
               acc_sh, idxb, rows, sg0, sg1, ss0, ss1, sx0, sx1, sx2, sx3):
    c = lax.axis_index("c")
    s = lax.axis_index("s")
    w = c * NS + s
    sg = [sg0, sg1]
    ss = [ss0, ss1]
    sx = [sx0, sx1, sx2, sx3]

    def xfire(i, q):
        pltpu.async_copy(e3_hbm.at[pl.ds(w * EPT + i * K, K)], idxb.at[q, 0],
                         sx[q])
        pltpu.async_copy(e3_hbm.at[pl.ds(NW * EPT + w * EPT + i * K, K)],
                         idxb.at[q, 1], sx[q])

    def xwait(q):
        pltpu.make_async_copy(e3_hbm.at[pl.ds(0, K)], idxb.at[q, 0],
                              sx[q]).wait()
        pltpu.make_async_copy(e3_hbm.at[pl.ds(0, K)], idxb.at[q, 1],
                              sx[q]).wait()

    def gfire(b, q):
        pltpu.async_copy(g_hbm.at[idxb.at[q, 0]], rows.at[b], sg[b])

    def gwait(b):
        pltpu.make_async_copy(g_hbm.at[idxb.at[0, 0]], rows.at[b],
                              sg[b]).wait()

    def sfire(b, q):
        pltpu.async_copy(rows.at[b], acc_sh.at[idxb.at[q, 1]], ss[b],
                         add=True)

    def swait(b):
        pltpu.make_async_copy(rows.at[b], acc_sh.at[idxb.at[0, 1]],
                              ss[b]).wait()

    pltpu.sync_copy(znd_hbm, acc_sh.at[pl.ds(s * RPT, RPT)])
    xfire(0, 0)
    plsc.subcore_barrier()

    def body(j, carry):
        for u in range(4):
            b = u % 2
            q = u
            qm = (u + 3) % 4

            if u >= 2:
                swait(b)
            else:
                @pl.when(j >= 1)
                def _():
                    swait(b)

            xwait(q)
            gfire(b, q)
            xfire(j * 4 + u + 1, (u + 1) % 4)

            if u >= 1:
                gwait(1 - b)
                sfire(1 - b, qm)
            else:
                @pl.when(j >= 1)
                def _():
                    gwait(1 - b)
                    sfire(1 - b, qm)
        return carry

    lax.fori_loop(0, (CHF - 2) // 4, body, 0)

    swait(0)
    xwait(0)
    gfire(0, 0)
    xfire(CHF - 1, 1)
    gwait(1)
    sfire(1, 3)

    swait(1)
    xwait(1)
    gfire(1, 1)
    for t in range((K - REM) // 16):
        vals = N + t * 16 + lax.iota(jnp.int32, 16)
        idxb[2, 0, pl.ds(REM + t * 16, 16)] = vals
        idxb[2, 1, pl.ds(REM + t * 16, 16)] = vals
    pltpu.async_copy(e3_hbm.at[pl.ds(w * EPT + CHF * K, REM)],
                     idxb.at[2, 0, pl.ds(0, REM)], sx[2])
    pltpu.async_copy(e3_hbm.at[pl.ds(NW * EPT + w * EPT + CHF * K, REM)],
                     idxb.at[2, 1, pl.ds(0, REM)], sx[2])
    gwait(0)
    sfire(0, 0)

    swait(0)
    pltpu.make_async_copy(e3_hbm.at[pl.ds(0, REM)],
                          idxb.at[2, 0, pl.ds(0, REM)], sx[2]).wait()
    pltpu.make_async_copy(e3_hbm.at[pl.ds(0, REM)],
                          idxb.at[2, 1, pl.ds(0, REM)], sx[2]).wait()
    gfire(0, 2)
    gwait(1)
    sfire(1, 1)

    gwait(0)
    sfire(0, 2)
    swait(1)
    swait(0)
    plsc.subcore_barrier()
    pltpu.sync_copy(acc_sh.at[pl.ds(s * RPT, RPT)],
                    acc_hbm.at[pl.ds(c * NP + s * RPT, RPT)])


def _dinv_mat(cnt_ref, i):
    cz = cnt_ref[...]
    deg = (cz[0] + cz[1]).reshape(1, RB) + 1.0
    lanes = lax.broadcasted_iota(jnp.int32, (1, RB), 1)
    valid = (i * RB + lanes) < N
    dv = jnp.where(valid, lax.rsqrt(deg), 0.0)
    ones = jnp.ones((1, D), jnp.float32)
    return lax.dot_general(dv, ones, (((0,), (0,)), ((), ())),
                           preferred_element_type=jnp.float32)


def _mm1_body(cnt_ref, x_ref, w_ref, g_ref):
    i = pl.program_id(0)
    dm = _dinv_mat(cnt_ref, i)
    xw = jnp.dot(x_ref[...], w_ref[...], preferred_element_type=jnp.float32)
    g_ref[...] = dm * xw


def _mm2_body(cnt_ref, acc_ref, g1_ref, w_ref, b_ref, g2_ref):
    i = pl.program_id(0)
    dm = _dinv_mat(cnt_ref, i)
    a = acc_ref[...]
    pre = dm * (a[0] + a[1] + g1_ref[...]) + b_ref[...]
    h = jnp.maximum(pre, 0.0)
    hw = jnp.dot(h, w_ref[...], preferred_element_type=jnp.float32)
    g2_ref[...] = dm * hw


def _fin_body(cnt_ref, acc_ref, g2_ref, b_ref, out_ref):
    i = pl.program_id(0)
    dm = _dinv_mat(cnt_ref, i)
    a = acc_ref[...]
    out_ref[...] = dm * (a[0] + a[1] + g2_ref[...]) + b_ref[...]


def kernel(x, edge_index, W1, b1, W2, b2):
    ei = edge_index.astype(jnp.int32)
    e3 = ei.reshape(2 * NW * EPT)
    xp = jnp.pad(x, ((0, NP - N), (0, 0)))
    ones_k = np.ones((KC,), np.float32)
    zn = np.zeros((RPT,), np.float32)
    znd = np.zeros((RPT, D), np.float32)

    mesh = plsc.VectorSubcoreMesh(core_axis_name="c", subcore_axis_name="s")

    cnt = pl.kernel(
        _count_body,
        out_type=jax.ShapeDtypeStruct((NC * NP,), jnp.float32),
        mesh=mesh,
        scratch_types=[
            pltpu.VMEM_SHARED((NP,), jnp.float32),
            pltpu.VMEM((CHC, KC), jnp.int32),
            pltpu.VMEM((KC,), jnp.float32),
            pltpu.SemaphoreType.DMA,
        ],
    )(e3, ones_k, zn)
    cntr = cnt.reshape(NC, NRB, 1, RB)

    edge_call = pl.kernel(
        _edge_body,
        out_type=jax.ShapeDtypeStruct((NC * NP, D), jnp.float32),
        mesh=mesh,
        scratch_types=[
            pltpu.VMEM_SHARED((NP, D), jnp.float32),
            pltpu.VMEM((4, 2, K), jnp.int32),
            pltpu.VMEM((2, K, D), jnp.float32),
        ] + [pltpu.SemaphoreType.DMA] * 8,
    )

    g1 = pl.pallas_call(
        _mm1_body,
        grid=(NRB,),
        in_specs=[
            pl.BlockSpec((NC, 1, 1, RB), lambda i: (0, i, 0, 0)),
            pl.BlockSpec((RB, D), lambda i: (i, 0)),
            pl.BlockSpec((D, D), lambda i: (0, 0)),
        ],
        out_specs=pl.BlockSpec((RB, D), lambda i: (i, 0)),
        out_shape=jax.ShapeDtypeStruct((NP, D), jnp.float32),
    )(cntr, xp, W1)

    acc1 = edge_call(g1, e3, znd).reshape(NC, NP, D)

    g2 = pl.pallas_call(
        _mm2_body,
        grid=(NRB,),
        in_specs=[
            pl.BlockSpec((NC, 1, 1, RB), lambda i: (0, i, 0, 0)),
            pl.BlockSpec((NC, RB, D), lambda i: (0, i, 0)),
            pl.BlockSpec((RB, D), lambda i: (i, 0)),
            pl.BlockSpec((D, D), lambda i: (0, 0)),
            pl.BlockSpec((1, D), lambda i: (0, 0)),
        ],
        out_specs=pl.BlockSpec((RB, D), lambda i: (i, 0)),
        out_shape=jax.ShapeDtypeStruct((NP, D), jnp.float32),
    )(cntr, acc1, g1, W2, b1.reshape(1, D))

    acc2 = edge_call(g2, e3, znd).reshape(NC, NP, D)

    out = pl.pallas_call(
        _fin_body,
        grid=(NRB,),
        in_specs=[
            pl.BlockSpec((NC, 1, 1, RB), lambda i: (0, i, 0, 0)),
            pl.BlockSpec((NC, RB, D), lambda i: (0, i, 0)),
            pl.BlockSpec((RB, D), lambda i: (i, 0)),
            pl.BlockSpec((1, D), lambda i: (0, 0)),
        ],
        out_specs=pl.BlockSpec((RB, D), lambda i: (i, 0)),
        out_shape=jax.ShapeDtypeStruct((N, D), jnp.float32),
    )(cntr, acc2, g2, b2.reshape(1, D))

    return out
